# baseline (speedup 1.0000x reference)
import functools

import jax
import jax.numpy as jnp
from jax import lax
from jax.experimental import pallas as pl
from jax.experimental.pallas import tpu as pltpu
from jax.experimental.pallas import tpu_sc as plsc

_REP = 8


def _sc_broadcast(pe_hbm, out_hbm, rep_v, sem):
    nc = 2
    wid = lax.axis_index("s") * nc + lax.axis_index("c")
    per_w = out_hbm.shape[0] // 32
    base = wid * per_w
    for r in range(_REP):
        pltpu.sync_copy(pe_hbm, rep_v.at[pl.ds(r, 1)])
    copies = [
        pltpu.async_copy(rep_v, out_hbm.at[pl.ds(base + j * _REP, _REP)], sem)
        for j in range(per_w // _REP)
    ]
    for c in copies:
        c.wait()


def kernel(x, pos_embed):
    batch = x.shape[0]
    max_len, d_model = pos_embed.shape
    row = max_len * d_model
    pe_flat = pos_embed.reshape(1, row)
    mesh = plsc.VectorSubcoreMesh(core_axis_name="c", subcore_axis_name="s")
    k = functools.partial(
        pl.kernel,
        mesh=mesh,
        out_type=jax.ShapeDtypeStruct((batch, row), jnp.float32),
        scratch_types=[
            pltpu.VMEM((_REP, row), jnp.float32),
            pltpu.SemaphoreType.DMA,
        ],
    )(_sc_broadcast)
    out = k(pe_flat)
    return out.reshape(batch, max_len, d_model)


# speedup vs baseline: 1.0417x; 1.0417x over previous
import functools

import jax
import jax.numpy as jnp
from jax import lax
from jax.experimental import pallas as pl
from jax.experimental.pallas import tpu as pltpu
from jax.experimental.pallas import tpu_sc as plsc

_REP = 8


def _sc_broadcast(pe8_hbm, out_hbm, rep_v, sem):
    nc = 2
    wid = lax.axis_index("s") * nc + lax.axis_index("c")
    per_w = out_hbm.shape[0] // 32
    base = wid * per_w
    pltpu.sync_copy(pe8_hbm, rep_v)
    copies = [
        pltpu.async_copy(rep_v, out_hbm.at[pl.ds(base + j * _REP, _REP)], sem)
        for j in range(per_w // _REP)
    ]
    for c in copies:
        c.wait()


def kernel(x, pos_embed):
    batch = x.shape[0]
    max_len, d_model = pos_embed.shape
    row = max_len * d_model
    pe8 = jnp.tile(pos_embed.reshape(1, row), (_REP, 1))
    mesh = plsc.VectorSubcoreMesh(core_axis_name="c", subcore_axis_name="s")
    k = functools.partial(
        pl.kernel,
        mesh=mesh,
        out_type=jax.ShapeDtypeStruct((batch, row), jnp.float32),
        scratch_types=[
            pltpu.VMEM((_REP, row), jnp.float32),
            pltpu.SemaphoreType.DMA,
        ],
        compiler_params=pltpu.CompilerParams(use_tc_tiling_on_sc=True),
    )(_sc_broadcast)
    out = k(pe8)
    return out.reshape(batch, max_len, d_model)
